# baseline (speedup 1.0000x reference)
import jax
import jax.numpy as jnp
from jax.experimental import pallas as pl
from jax.experimental.pallas import tpu as pltpu

NBR = 5
B_PER_BLOCK = 5


def _body(x_ref, n_ref, wb_ref, wl_ref, wr_ref, o_ref):
    bb, j, d = x_ref.shape
    r = bb * j
    agg = n_ref[:, 0, :, :] * wb_ref[0, :]
    for k in range(1, NBR):
        agg = agg + n_ref[:, k, :, :] * wb_ref[k, :]
    o_ref[...] = (
        jnp.dot(x_ref[...].reshape(r, d), wl_ref[...],
                preferred_element_type=jnp.float32)
        + jnp.dot(agg.reshape(r, d), wr_ref[...],
                  preferred_element_type=jnp.float32)
    ).reshape(bb, j, d)


def kernel(x, neigh_x, w_aggr1, W_l, W_r):
    b, j, d = x.shape
    n_rows = b * j
    nt = jnp.transpose(neigh_x, (0, 2, 1, 3))
    wb = jnp.pad(
        jnp.broadcast_to(w_aggr1[0][:, None], (NBR, d)), ((0, 8 - NBR), (0, 0))
    )
    wl_t = W_l.T
    wr_t = W_r.T

    bb = B_PER_BLOCK
    grid = (b // bb,)
    out = pl.pallas_call(
        _body,
        grid=grid,
        in_specs=[
            pl.BlockSpec((bb, j, d), lambda i: (i, 0, 0)),
            pl.BlockSpec((bb, NBR, j, d), lambda i: (i, 0, 0, 0)),
            pl.BlockSpec((8, d), lambda i: (0, 0)),
            pl.BlockSpec((d, d), lambda i: (0, 0)),
            pl.BlockSpec((d, d), lambda i: (0, 0)),
        ],
        out_specs=pl.BlockSpec((bb, j, d), lambda i: (i, 0, 0)),
        out_shape=jax.ShapeDtypeStruct((b, j, d), jnp.float32),
        compiler_params=pltpu.CompilerParams(
            dimension_semantics=("arbitrary",),
        ),
    )(x, nt, wb, wl_t, wr_t)
    return out.reshape(n_rows, d)


# speedup vs baseline: 1.2278x; 1.2278x over previous
import jax
import jax.numpy as jnp
from jax.experimental import pallas as pl
from jax.experimental.pallas import tpu as pltpu

NBR = 5
B_PER_BLOCK = 10


def _body(x_ref, n_ref, wb_ref, wl_ref, wr_ref, o_ref):
    bb, j, d = x_ref.shape
    r = bb * j
    agg = n_ref[:, 0, :, :] * wb_ref[0, :]
    for k in range(1, NBR):
        agg = agg + n_ref[:, k, :, :] * wb_ref[k, :]
    o_ref[...] = (
        jnp.dot(x_ref[...].reshape(r, d), wl_ref[...],
                preferred_element_type=jnp.float32)
        + jnp.dot(agg.reshape(r, d), wr_ref[...],
                  preferred_element_type=jnp.float32)
    ).reshape(bb, j, d)


def kernel(x, neigh_x, w_aggr1, W_l, W_r):
    b, j, d = x.shape
    n_rows = b * j
    nt = jnp.transpose(neigh_x, (0, 2, 1, 3))
    wb = jnp.pad(
        jnp.broadcast_to(w_aggr1[0][:, None], (NBR, d)), ((0, 8 - NBR), (0, 0))
    )
    wl_t = W_l.T
    wr_t = W_r.T

    bb = B_PER_BLOCK
    grid = (b // bb,)
    out = pl.pallas_call(
        _body,
        grid=grid,
        in_specs=[
            pl.BlockSpec((bb, j, d), lambda i: (i, 0, 0)),
            pl.BlockSpec((bb, NBR, j, d), lambda i: (i, 0, 0, 0)),
            pl.BlockSpec((8, d), lambda i: (0, 0)),
            pl.BlockSpec((d, d), lambda i: (0, 0)),
            pl.BlockSpec((d, d), lambda i: (0, 0)),
        ],
        out_specs=pl.BlockSpec((bb, j, d), lambda i: (i, 0, 0)),
        out_shape=jax.ShapeDtypeStruct((b, j, d), jnp.float32),
        compiler_params=pltpu.CompilerParams(
            dimension_semantics=("arbitrary",),
        ),
    )(x, nt, wb, wl_t, wr_t)
    return out.reshape(n_rows, d)


# speedup vs baseline: 1.3260x; 1.0800x over previous
import jax
import jax.numpy as jnp
from jax import lax
from jax.experimental import pallas as pl
from jax.experimental.pallas import tpu as pltpu

NBR = 5
B_PER_BLOCK = 10


def _body(x_ref, n_ref, w_ref, wl_ref, wr_ref, o_ref):
    bb, j, d = x_ref.shape
    r = bb * j
    agg = n_ref[:, 0, :, :] * w_ref[0, 0]
    for k in range(1, NBR):
        agg = agg + n_ref[:, k, :, :] * w_ref[0, k]
    dims = (((1,), (1,)), ((), ()))
    o_ref[...] = (
        lax.dot_general(x_ref[...].reshape(r, d), wl_ref[...], dims,
                        preferred_element_type=jnp.float32)
        + lax.dot_general(agg.reshape(r, d), wr_ref[...], dims,
                          preferred_element_type=jnp.float32)
    ).reshape(bb, j, d)


def kernel(x, neigh_x, w_aggr1, W_l, W_r):
    b, j, d = x.shape
    n_rows = b * j
    nt = jnp.transpose(neigh_x, (0, 2, 1, 3))

    bb = B_PER_BLOCK
    grid = (b // bb,)
    out = pl.pallas_call(
        _body,
        grid=grid,
        in_specs=[
            pl.BlockSpec((bb, j, d), lambda i: (i, 0, 0)),
            pl.BlockSpec((bb, NBR, j, d), lambda i: (i, 0, 0, 0)),
            pl.BlockSpec((1, NBR), lambda i: (0, 0), memory_space=pltpu.SMEM),
            pl.BlockSpec((d, d), lambda i: (0, 0)),
            pl.BlockSpec((d, d), lambda i: (0, 0)),
        ],
        out_specs=pl.BlockSpec((bb, j, d), lambda i: (i, 0, 0)),
        out_shape=jax.ShapeDtypeStruct((b, j, d), jnp.float32),
        compiler_params=pltpu.CompilerParams(
            dimension_semantics=("arbitrary",),
        ),
    )(x, nt, w_aggr1, W_l, W_r)
    return out.reshape(n_rows, d)
